# baseline (speedup 1.0000x reference)
import functools

import jax
import jax.numpy as jnp
from jax import lax
from jax.experimental import pallas as pl
from jax.experimental.pallas import tpu as pltpu
from jax.experimental.pallas import tpu_sc as plsc

_NC = 2
_NS = 16
_LANES = 16
_W = 2560


def _npad(N):
    return (-(-N // (_NS * 128)) * 128) * _NS


@functools.partial(jax.jit, static_argnums=(2, 3))
def _segment_partials(z, edge_index, N, E):
    NW = _NC * _NS
    G = E // _W
    assert G * _W == E
    NFULL, NEXTRA = divmod(G, NW)
    NPAD = _npad(N)
    SL = NPAD // _NS

    mesh = plsc.VectorSubcoreMesh(core_axis_name="c", subcore_axis_name="s")

    @functools.partial(
        pl.kernel,
        out_type=jax.ShapeDtypeStruct((_NC * NPAD,), jnp.float32),
        mesh=mesh,
        compiler_params=pltpu.CompilerParams(
            use_tc_tiling_on_sc=True, needs_layout_passes=False),
        scratch_types=[
            pltpu.VMEM_SHARED((NPAD,), jnp.float32),
            pltpu.VMEM((N,), jnp.float32),
            [pltpu.VMEM((2, _W), jnp.int32)] * 2,
            [pltpu.VMEM((_W,), jnp.int32)] * 2,
            [pltpu.VMEM((_W,), jnp.float32)] * 2,
            [pltpu.SemaphoreType.DMA] * 2,
            [pltpu.SemaphoreType.DMA] * 2,
        ],
    )
    def seg(z_hbm, ei_hbm, out_hbm,
            acc_sh, z_t, ei_v, src_v, val_v, esem, ssem):
        cid = lax.axis_index("c")
        sid = lax.axis_index("s")
        wid = cid * _NS + sid
        nwin = jnp.where(wid < NEXTRA, NFULL + 1, NFULL)

        def fetch(i, s):
            off = (wid + i * NW) * _W
            pltpu.async_copy(ei_hbm.at[:, pl.ds(off, _W)], ei_v[s], esem[s])

        fetch(0, 0)
        fetch(1, 1)

        def zbody(i, carry):
            val_v[0][pl.ds(i * _LANES, _LANES)] = jnp.zeros((_LANES,), jnp.float32)
            return carry

        lax.fori_loop(0, _W // _LANES, zbody, 0)
        full, rem = divmod(SL, _W)
        for q in range(full):
            pltpu.sync_copy(val_v[0], acc_sh.at[pl.ds(sid * SL + q * _W, _W)])
        if rem:
            pltpu.sync_copy(val_v[0].at[pl.ds(0, rem)],
                            acc_sh.at[pl.ds(sid * SL + full * _W, rem)])

        pltpu.sync_copy(z_hbm, z_t)

        plsc.subcore_barrier()

        def window(i, s, first):
            pltpu.make_async_copy(ei_hbm.at[:, pl.ds(0, _W)],
                                  ei_v[s], esem[s]).wait()
            if not first:
                pltpu.make_async_copy(val_v[s], acc_sh.at[src_v[s]],
                                      ssem[s]).wait()

            @plsc.parallel_loop(0, _W, step=_LANES, unroll=16)
            def gbody(k):
                src_v[s][pl.ds(k, _LANES)] = ei_v[s][0, pl.ds(k, _LANES)]
                idx = ei_v[s][1, pl.ds(k, _LANES)]
                val_v[s][pl.ds(k, _LANES)] = plsc.load_gather(z_t, [idx])

            @pl.when(i + 2 < nwin)
            def _():
                fetch(i + 2, s)

            pltpu.async_copy(val_v[s], acc_sh.at[src_v[s]], ssem[s], add=True)

        window(0, 0, True)
        window(1, 1, True)

        def body(j, carry):
            window(2 * j, 0, False)
            window(2 * j + 1, 1, False)
            return carry

        lax.fori_loop(1, nwin // 2, body, 0)

        @pl.when(nwin % 2 == 1)
        def _odd_tail():
            pltpu.make_async_copy(ei_hbm.at[:, pl.ds(0, _W)],
                                  ei_v[0], esem[0]).wait()
            pltpu.make_async_copy(val_v[0], acc_sh.at[src_v[0]], ssem[0]).wait()

            @plsc.parallel_loop(0, _W, step=_LANES, unroll=16)
            def gbody(k):
                src_v[0][pl.ds(k, _LANES)] = ei_v[0][0, pl.ds(k, _LANES)]
                idx = ei_v[0][1, pl.ds(k, _LANES)]
                val_v[0][pl.ds(k, _LANES)] = plsc.load_gather(z_t, [idx])

            pltpu.async_copy(val_v[0], acc_sh.at[src_v[0]], ssem[0], add=True)

        for h in range(2):
            pltpu.make_async_copy(val_v[h], acc_sh.at[src_v[h]], ssem[h]).wait()

        plsc.subcore_barrier()

        @pl.when(sid == 0)
        def _writeout():
            pltpu.sync_copy(acc_sh, out_hbm.at[pl.ds(cid * NPAD, NPAD)])

    return seg(z, edge_index)


def _make_combine(N, NPAD):
    def body(s_ref, z_ref, d_ref, p_ref, o_ref):
        alpha = jnp.exp(s_ref[0])
        beta = -alpha * jnp.exp(s_ref[1])
        gamma = jnp.exp(s_ref[2])
        bias = s_ref[3]
        logd = jnp.log(d_ref[...])
        az = p_ref[pl.ds(0, N)] + p_ref[pl.ds(NPAD, N)]
        o_ref[...] = (alpha * jnp.exp(gamma * logd) * z_ref[...]
                      + beta * jnp.exp((gamma - 1.0) * logd) * az + bias)

    return pl.pallas_call(
        body,
        out_shape=jax.ShapeDtypeStruct((N,), jnp.float32),
        in_specs=[
            pl.BlockSpec(memory_space=pltpu.SMEM),
            pl.BlockSpec(memory_space=pltpu.VMEM),
            pl.BlockSpec(memory_space=pltpu.VMEM),
            pl.BlockSpec(memory_space=pltpu.VMEM),
        ],
        out_specs=pl.BlockSpec(memory_space=pltpu.VMEM),
    )


def kernel(z, edge_index, D, params):
    N = z.shape[0]
    E = edge_index.shape[1]

    partial = _segment_partials(z, edge_index, N, E)
    NPAD = _npad(N)

    return _make_combine(N, NPAD)(params, z, D, partial)
